# baseline (speedup 1.0000x reference)
import functools

import jax
import jax.numpy as jnp
from jax import lax
from jax.experimental import pallas as pl
from jax.experimental.pallas import tpu as pltpu
from jax.experimental.pallas import tpu_sc as plsc

_NUM_SC_CORES = 2
_NUM_SC_SUBCORES = 16
_GATHER_CHUNK = 32


def _sc_gather(table, idx):
    n_rows, row_dim = idx.shape[0], table.shape[1]
    n_workers = _NUM_SC_CORES * _NUM_SC_SUBCORES
    per_worker = n_rows // n_workers
    chunk = min(_GATHER_CHUNK, per_worker)
    n_chunks = per_worker // chunk

    mesh = plsc.VectorSubcoreMesh(core_axis_name="c", subcore_axis_name="s")

    @functools.partial(
        pl.kernel,
        mesh=mesh,
        out_type=jax.ShapeDtypeStruct((n_rows, row_dim), table.dtype),
        scratch_types=[
            pltpu.VMEM((per_worker,), jnp.int32),
            pltpu.VMEM((chunk, row_dim), table.dtype),
            pltpu.SemaphoreType.DMA,
        ],
    )
    def gather_kernel(table_hbm, idx_hbm, out_hbm, idx_v, rows_v, sem):
        wid = lax.axis_index("s") * _NUM_SC_CORES + lax.axis_index("c")
        base = wid * per_worker
        pltpu.sync_copy(idx_hbm.at[pl.ds(base, per_worker)], idx_v)

        @pl.loop(0, n_chunks)
        def _(c):
            off = c * chunk
            pltpu.async_copy(
                table_hbm.at[idx_v.at[pl.ds(off, chunk)]], rows_v, sem
            ).wait()
            pltpu.sync_copy(rows_v, out_hbm.at[pl.ds(base + off, chunk)])

    return gather_kernel(table, idx)


def _mm_body(x_ref, w_ref, o_ref, wbf_ref):
    @pl.when(pl.program_id(2) == 0)
    def _():
        wbf_ref[...] = w_ref[0].astype(jnp.bfloat16)

    xb = x_ref[...].astype(jnp.bfloat16)
    o_ref[0, 0] = lax.dot_general(
        xb,
        wbf_ref[...],
        (((1,), (0,)), ((), ())),
        preferred_element_type=jnp.float32,
    )


def kernel(X, ind, W):
    B, T, I = X.shape
    E, _, J = W.shape
    K = ind.shape[2]
    N = B * E * K
    BK = 512
    KB = K // BK

    X_flat = X.reshape(B * T, I)
    offset = (jnp.arange(B, dtype=jnp.int32) * T).reshape(B, 1, 1)
    idx = (ind.astype(jnp.int32) + offset).reshape(N)

    x_gathered = _sc_gather(X_flat, idx)

    out = pl.pallas_call(
        _mm_body,
        grid=(B, E, KB),
        in_specs=[
            pl.BlockSpec((BK, I), lambda b, e, k: ((b * E + e) * KB + k, 0)),
            pl.BlockSpec((1, I, J), lambda b, e, k: (e, 0, 0)),
        ],
        out_specs=pl.BlockSpec((1, 1, BK, J), lambda b, e, k: (b, e, k, 0)),
        out_shape=jax.ShapeDtypeStruct((B, E, K, J), jnp.float32),
        scratch_shapes=[pltpu.VMEM((I, J), jnp.bfloat16)],
    )(x_gathered, W)
    return out
